# initial kernel scaffold (unmeasured)
import jax
import jax.numpy as jnp
from jax import lax
from jax.experimental import pallas as pl
from jax.experimental.pallas import tpu as pltpu

N_DEV = 4


def kernel(x, dest):
    n_rows, n_cols = x.shape
    dest = dest.astype(jnp.int32)

    onehot = (dest[:, None] == jnp.arange(N_DEV, dtype=jnp.int32)[None, :])
    onehot = onehot.astype(jnp.int32)
    cnt = onehot.sum(axis=0, dtype=jnp.int32)
    rank = jnp.take_along_axis(
        jnp.cumsum(onehot, axis=0) - 1, dest[:, None], axis=1
    )[:, 0].astype(jnp.int32)
    cnt_pad = jnp.zeros((128,), jnp.int32).at[:N_DEV].set(cnt)

    def body(x_ref, dest_ref, rank_ref, cnt_ref, out_ref,
             xb_ref, cnt_all_ref, cnt_sm_ref, base_ref,
             cnt_send_sem, cnt_recv_sem, send_sem, recv_sem, local_sem):
        my = lax.axis_index("i")

        bar = pltpu.get_barrier_semaphore()
        for k in range(1, N_DEV):
            pl.semaphore_signal(
                bar, inc=1, device_id=((my + k) % N_DEV,),
                device_id_type=pl.DeviceIdType.MESH,
            )
        pl.semaphore_wait(bar, N_DEV - 1)

        cnt_all_ref[pl.ds(my, 1), :] = cnt_ref[:][None, :]
        for k in range(1, N_DEV):
            pltpu.make_async_remote_copy(
                src_ref=cnt_all_ref.at[my],
                dst_ref=cnt_all_ref.at[my],
                send_sem=cnt_send_sem, recv_sem=cnt_recv_sem,
                device_id=((my + k) % N_DEV,),
                device_id_type=pl.DeviceIdType.MESH,
            ).start()

        xb_ref[...] = x_ref[...].astype(jnp.bfloat16)

        cnt_dummy = pltpu.make_async_remote_copy(
            src_ref=cnt_all_ref.at[0], dst_ref=cnt_all_ref.at[0],
            send_sem=cnt_send_sem, recv_sem=cnt_recv_sem,
            device_id=(my,), device_id_type=pl.DeviceIdType.MESH,
        )
        for _ in range(N_DEV - 1):
            cnt_dummy.wait_send()
        for _ in range(N_DEV - 1):
            cnt_dummy.wait_recv()

        cp = pltpu.make_async_copy(cnt_all_ref, cnt_sm_ref, local_sem)
        cp.start()
        cp.wait()

        n_self = jnp.int32(0)
        for d in range(N_DEV):
            acc = jnp.int32(0)
            for s in range(N_DEV):
                acc = acc + jnp.where(my > s, cnt_sm_ref[s, d], 0)
            base_ref[d] = acc
            n_self = n_self + jnp.where(my == d, cnt_sm_ref[d, d], 0)

        def send_row(j, carry):
            dd = dest_ref[j]
            oo = base_ref[dd] + rank_ref[j]
            row = pltpu.make_async_remote_copy(
                src_ref=xb_ref.at[pl.ds(j, 1)],
                dst_ref=out_ref.at[pl.ds(oo, 1)],
                send_sem=send_sem, recv_sem=recv_sem,
                device_id=(dd,), device_id_type=pl.DeviceIdType.MESH,
            )

            @pl.when(dd != my)
            def _():
                row.start()

            @pl.when(dd == my)
            def _():
                out_ref[pl.ds(oo, 1), :] = xb_ref[pl.ds(j, 1), :]

            return carry

        lax.fori_loop(0, n_rows, send_row, 0)

        n_remote = n_rows - n_self
        dummy = pltpu.make_async_remote_copy(
            src_ref=xb_ref.at[pl.ds(0, 1)],
            dst_ref=out_ref.at[pl.ds(0, 1)],
            send_sem=send_sem, recv_sem=recv_sem,
            device_id=(my,), device_id_type=pl.DeviceIdType.MESH,
        )

        def drain_send(i, c):
            dummy.wait_send()
            return c

        def drain_recv(i, c):
            dummy.wait_recv()
            return c

        lax.fori_loop(0, n_remote, drain_send, 0)
        lax.fori_loop(0, n_remote, drain_recv, 0)

    return pl.pallas_call(
        body,
        out_shape=jax.ShapeDtypeStruct((n_rows, n_cols), jnp.bfloat16),
        in_specs=[
            pl.BlockSpec(memory_space=pltpu.VMEM),
            pl.BlockSpec(memory_space=pltpu.SMEM),
            pl.BlockSpec(memory_space=pltpu.SMEM),
            pl.BlockSpec(memory_space=pltpu.VMEM),
        ],
        out_specs=pl.BlockSpec(memory_space=pltpu.VMEM),
        scratch_shapes=[
            pltpu.VMEM((n_rows, n_cols), jnp.bfloat16),
            pltpu.VMEM((N_DEV, 128), jnp.int32),
            pltpu.SMEM((N_DEV, 128), jnp.int32),
            pltpu.SMEM((N_DEV,), jnp.int32),
            pltpu.SemaphoreType.DMA,
            pltpu.SemaphoreType.DMA,
            pltpu.SemaphoreType.DMA,
            pltpu.SemaphoreType.DMA,
            pltpu.SemaphoreType.DMA,
        ],
        compiler_params=pltpu.CompilerParams(collective_id=0),
    )(x, dest, rank, cnt_pad)


# baseline (device time: 53616 ns/iter reference)
import jax
import jax.numpy as jnp
from jax import lax
from jax.experimental import pallas as pl
from jax.experimental.pallas import tpu as pltpu

N_DEV = 4
SUB = 4
LANE = 128


def kernel(x, dest):
    n_rows, n_cols = x.shape
    assert n_cols == SUB * LANE
    dest = dest.astype(jnp.int32)

    onehot = (dest[:, None] == jnp.arange(N_DEV, dtype=jnp.int32)[None, :])
    onehot = onehot.astype(jnp.int32)
    cnt = onehot.sum(axis=0, dtype=jnp.int32)
    rank = jnp.take_along_axis(
        jnp.cumsum(onehot, axis=0) - 1, dest[:, None], axis=1
    )[:, 0].astype(jnp.int32)
    cnt_pad = jnp.zeros((1, LANE), jnp.int32).at[0, :N_DEV].set(cnt)
    x3 = x.reshape(n_rows, SUB, LANE)

    def body(x_ref, dest_ref, rank_ref, cnt_ref, out_ref,
             xb_ref, cnt_all_ref, cnt_sm_ref, base_ref,
             cnt_send_sem, cnt_recv_sem, send_sem, recv_sem, local_sem):
        my = lax.axis_index("i")

        bar = pltpu.get_barrier_semaphore()
        for k in range(1, N_DEV):
            pl.semaphore_signal(
                bar, inc=1, device_id=((my + k) % N_DEV,),
                device_id_type=pl.DeviceIdType.MESH,
            )
        pl.semaphore_wait(bar, N_DEV - 1)

        cnt_all_ref[pl.ds(my, 1)] = cnt_ref[:][None]
        for k in range(1, N_DEV):
            pltpu.make_async_remote_copy(
                src_ref=cnt_all_ref.at[pl.ds(my, 1)],
                dst_ref=cnt_all_ref.at[pl.ds(my, 1)],
                send_sem=cnt_send_sem, recv_sem=cnt_recv_sem,
                device_id=((my + k) % N_DEV,),
                device_id_type=pl.DeviceIdType.MESH,
            ).start()

        xb_ref[...] = x_ref[...].astype(jnp.bfloat16)

        cnt_dummy = pltpu.make_async_remote_copy(
            src_ref=cnt_all_ref.at[pl.ds(0, 1)],
            dst_ref=cnt_all_ref.at[pl.ds(0, 1)],
            send_sem=cnt_send_sem, recv_sem=cnt_recv_sem,
            device_id=(my,), device_id_type=pl.DeviceIdType.MESH,
        )
        for _ in range(N_DEV - 1):
            cnt_dummy.wait_send()
        for _ in range(N_DEV - 1):
            cnt_dummy.wait_recv()

        cp = pltpu.make_async_copy(cnt_all_ref, cnt_sm_ref, local_sem)
        cp.start()
        cp.wait()

        n_self = jnp.int32(0)
        for d in range(N_DEV):
            acc = jnp.int32(0)
            for s in range(N_DEV):
                acc = acc + jnp.where(my > s, cnt_sm_ref[s, 0, d], 0)
            base_ref[d] = acc
            n_self = n_self + jnp.where(my == d, cnt_sm_ref[d, 0, d], 0)

        def send_row(j, carry):
            dd = dest_ref[j]
            oo = base_ref[dd] + rank_ref[j]
            row = pltpu.make_async_remote_copy(
                src_ref=xb_ref.at[pl.ds(j, 1)],
                dst_ref=out_ref.at[pl.ds(oo, 1)],
                send_sem=send_sem, recv_sem=recv_sem,
                device_id=(dd,), device_id_type=pl.DeviceIdType.MESH,
            )

            @pl.when(dd != my)
            def _():
                row.start()

            @pl.when(dd == my)
            def _():
                out_ref[pl.ds(oo, 1)] = xb_ref[pl.ds(j, 1)]

            return carry

        lax.fori_loop(0, n_rows, send_row, 0)

        n_remote = n_rows - n_self
        dummy = pltpu.make_async_remote_copy(
            src_ref=xb_ref.at[pl.ds(0, 1)],
            dst_ref=out_ref.at[pl.ds(0, 1)],
            send_sem=send_sem, recv_sem=recv_sem,
            device_id=(my,), device_id_type=pl.DeviceIdType.MESH,
        )

        def drain_send(i, c):
            dummy.wait_send()
            return c

        def drain_recv(i, c):
            dummy.wait_recv()
            return c

        lax.fori_loop(0, n_remote, drain_send, 0)
        lax.fori_loop(0, n_remote, drain_recv, 0)

    out3 = pl.pallas_call(
        body,
        out_shape=jax.ShapeDtypeStruct((n_rows, SUB, LANE), jnp.bfloat16),
        in_specs=[
            pl.BlockSpec(memory_space=pltpu.VMEM),
            pl.BlockSpec(memory_space=pltpu.SMEM),
            pl.BlockSpec(memory_space=pltpu.SMEM),
            pl.BlockSpec(memory_space=pltpu.VMEM),
        ],
        out_specs=pl.BlockSpec(memory_space=pltpu.VMEM),
        scratch_shapes=[
            pltpu.VMEM((n_rows, SUB, LANE), jnp.bfloat16),
            pltpu.VMEM((N_DEV, 1, LANE), jnp.int32),
            pltpu.SMEM((N_DEV, 1, LANE), jnp.int32),
            pltpu.SMEM((N_DEV,), jnp.int32),
            pltpu.SemaphoreType.DMA,
            pltpu.SemaphoreType.DMA,
            pltpu.SemaphoreType.DMA,
            pltpu.SemaphoreType.DMA,
            pltpu.SemaphoreType.DMA,
        ],
        compiler_params=pltpu.CompilerParams(collective_id=0),
    )(x3, dest, rank, cnt_pad)
    return out3.reshape(n_rows, n_cols)


# device time: 26523 ns/iter; 2.0215x vs baseline; 2.0215x over previous
import jax
import jax.numpy as jnp
from jax import lax
from jax.experimental import pallas as pl
from jax.experimental.pallas import tpu as pltpu

N_DEV = 4
SUB = 4
LANE = 128
MAX_BIT = 10


def kernel(x, dest):
    n_rows, n_cols = x.shape
    assert n_cols == SUB * LANE
    dest = dest.astype(jnp.int32)

    order = jnp.argsort(dest, stable=True)
    xs = x[order].astype(jnp.bfloat16).reshape(n_rows, SUB, LANE)
    cnt = jnp.sum(
        dest[:, None] == jnp.arange(N_DEV, dtype=jnp.int32)[None, :],
        axis=0, dtype=jnp.int32,
    )
    cnt_pad = jnp.zeros((1, LANE), jnp.int32).at[0, :N_DEV].set(cnt)

    def body(xs_ref, cnt_ref, out_ref,
             cnt_all_ref, cnt_sm_ref, myc_ref, loff_ref, base_ref, incc_ref,
             cnt_send_sem, cnt_recv_sem, send_sem, recv_sem, local_sem):
        my = lax.axis_index("i")

        bar = pltpu.get_barrier_semaphore()
        for k in range(1, N_DEV):
            pl.semaphore_signal(
                bar, inc=1, device_id=((my + k) % N_DEV,),
                device_id_type=pl.DeviceIdType.MESH,
            )
        pl.semaphore_wait(bar, N_DEV - 1)

        cnt_all_ref[pl.ds(my, 1)] = cnt_ref[:][None]
        for k in range(1, N_DEV):
            pltpu.make_async_remote_copy(
                src_ref=cnt_all_ref.at[pl.ds(my, 1)],
                dst_ref=cnt_all_ref.at[pl.ds(my, 1)],
                send_sem=cnt_send_sem, recv_sem=cnt_recv_sem,
                device_id=((my + k) % N_DEV,),
                device_id_type=pl.DeviceIdType.MESH,
            ).start()
        cnt_dummy = pltpu.make_async_remote_copy(
            src_ref=cnt_all_ref.at[pl.ds(0, 1)],
            dst_ref=cnt_all_ref.at[pl.ds(0, 1)],
            send_sem=cnt_send_sem, recv_sem=cnt_recv_sem,
            device_id=(my,), device_id_type=pl.DeviceIdType.MESH,
        )
        for _ in range(N_DEV - 1):
            cnt_dummy.wait_send()
        for _ in range(N_DEV - 1):
            cnt_dummy.wait_recv()

        cp = pltpu.make_async_copy(cnt_all_ref, cnt_sm_ref, local_sem)
        cp.start()
        cp.wait()

        for d in range(N_DEV):
            m = jnp.int32(0)
            lo = jnp.int32(0)
            b = jnp.int32(0)
            for s in range(N_DEV):
                c_sd = cnt_sm_ref[s, 0, d]
                m = m + jnp.where(my == s, c_sd, 0)
                b = b + jnp.where(my > s, c_sd, 0)
            for dp in range(d):
                for s in range(N_DEV):
                    lo = lo + jnp.where(my == s, cnt_sm_ref[s, 0, dp], 0)
            myc_ref[d] = m
            loff_ref[d] = lo
            base_ref[d] = b
        for s in range(N_DEV):
            ic = jnp.int32(0)
            for d in range(N_DEV):
                ic = ic + jnp.where(my == d, cnt_sm_ref[s, 0, d], 0)
            incc_ref[s] = ic

        def chunk_off(c, sz):
            return c & jnp.int32(~(2 * sz - 1))

        c_loc = jnp.int32(0)
        for d in range(N_DEV):
            c_loc = c_loc + jnp.where(my == d, cnt_sm_ref[d, 0, d], 0)
        lo_my = loff_ref[my]
        b_my = base_ref[my]
        for bit in range(MAX_BIT, -1, -1):
            sz = 1 << bit

            @pl.when((c_loc & sz) != 0)
            def _(sz=sz):
                off = chunk_off(c_loc, sz)
                pltpu.make_async_copy(
                    xs_ref.at[pl.ds(lo_my + off, sz)],
                    out_ref.at[pl.ds(b_my + off, sz)],
                    local_sem,
                ).start()

        for k in range(1, N_DEV):
            dd = (my + k) % N_DEV
            c = myc_ref[dd]
            s0 = loff_ref[dd]
            r0 = base_ref[dd]
            for bit in range(MAX_BIT, -1, -1):
                sz = 1 << bit

                @pl.when((c & sz) != 0)
                def _(sz=sz, c=c, s0=s0, r0=r0, dd=dd):
                    off = chunk_off(c, sz)
                    pltpu.make_async_remote_copy(
                        src_ref=xs_ref.at[pl.ds(s0 + off, sz)],
                        dst_ref=out_ref.at[pl.ds(r0 + off, sz)],
                        send_sem=send_sem, recv_sem=recv_sem,
                        device_id=(dd,),
                        device_id_type=pl.DeviceIdType.MESH,
                    ).start()

        def dummy_for(sz):
            return pltpu.make_async_remote_copy(
                src_ref=xs_ref.at[pl.ds(0, sz)],
                dst_ref=out_ref.at[pl.ds(0, sz)],
                send_sem=send_sem, recv_sem=recv_sem,
                device_id=(my,), device_id_type=pl.DeviceIdType.MESH,
            )

        for bit in range(MAX_BIT, -1, -1):
            sz = 1 << bit

            @pl.when((c_loc & sz) != 0)
            def _(sz=sz):
                pltpu.make_async_copy(
                    xs_ref.at[pl.ds(0, sz)],
                    out_ref.at[pl.ds(0, sz)],
                    local_sem,
                ).wait()

        for k in range(1, N_DEV):
            c = myc_ref[(my + k) % N_DEV]
            for bit in range(MAX_BIT, -1, -1):
                sz = 1 << bit

                @pl.when((c & sz) != 0)
                def _(sz=sz):
                    dummy_for(sz).wait_send()

        for k in range(1, N_DEV):
            c = incc_ref[(my + k) % N_DEV]
            for bit in range(MAX_BIT, -1, -1):
                sz = 1 << bit

                @pl.when((c & sz) != 0)
                def _(sz=sz):
                    dummy_for(sz).wait_recv()

    out3 = pl.pallas_call(
        body,
        out_shape=jax.ShapeDtypeStruct((n_rows, SUB, LANE), jnp.bfloat16),
        in_specs=[
            pl.BlockSpec(memory_space=pltpu.VMEM),
            pl.BlockSpec(memory_space=pltpu.VMEM),
        ],
        out_specs=pl.BlockSpec(memory_space=pltpu.VMEM),
        scratch_shapes=[
            pltpu.VMEM((N_DEV, 1, LANE), jnp.int32),
            pltpu.SMEM((N_DEV, 1, LANE), jnp.int32),
            pltpu.SMEM((N_DEV,), jnp.int32),
            pltpu.SMEM((N_DEV,), jnp.int32),
            pltpu.SMEM((N_DEV,), jnp.int32),
            pltpu.SMEM((N_DEV,), jnp.int32),
            pltpu.SemaphoreType.DMA,
            pltpu.SemaphoreType.DMA,
            pltpu.SemaphoreType.DMA,
            pltpu.SemaphoreType.DMA,
            pltpu.SemaphoreType.DMA,
        ],
        compiler_params=pltpu.CompilerParams(collective_id=0),
    )(xs, cnt_pad)
    return out3.reshape(n_rows, n_cols)
